# baseline (speedup 1.0000x reference)
import functools

import jax
import jax.numpy as jnp
from jax import lax
from jax.experimental import pallas as pl
from jax.experimental.pallas import tpu as pltpu
from jax.experimental.pallas import tpu_sc as plsc

W = 1280
H = 720
HP = 768
WHP = W * HP
BIG = float(1 << 22)

NC = 2
NS = 16
NW = NC * NS

CHUNK = 4096
ROWS = CHUNK // 128
SLICE = WHP // NS
ZCH = 7680


def _sc_scatter(events_x, events_y, events_polarity):
    n = events_x.shape[0]
    pw = n // NW
    nch = -(-pw // CHUNK)
    last_off = pw - CHUNK

    mesh = plsc.VectorSubcoreMesh(core_axis_name="c", subcore_axis_name="s")

    @functools.partial(
        pl.kernel,
        mesh=mesh,
        out_type=jax.ShapeDtypeStruct((NC, W, HP), jnp.float32),
        scratch_types=[
            pltpu.VMEM((2 * CHUNK,), jnp.int32),
            pltpu.VMEM((2 * CHUNK,), jnp.int32),
            pltpu.VMEM((2 * CHUNK,), jnp.int32),
            pltpu.VMEM((ROWS, 128), jnp.int32),
            pltpu.VMEM((ROWS, 128), jnp.float32),
            pltpu.VMEM((ROWS, 128), jnp.int32),
            pltpu.VMEM((ROWS, 128), jnp.float32),
            pltpu.VMEM((ZCH,), jnp.float32),
            pltpu.VMEM_SHARED((WHP,), jnp.float32),
            pltpu.SemaphoreType.DMA,
            pltpu.SemaphoreType.DMA,
            pltpu.SemaphoreType.DMA,
        ],
    )
    def body(exr, eyr, epr, outr, xv, yv, pv, idxb0, valb0, idxb1, valb1,
             zbuf, acc, sem_ev, sem_sc0, sem_sc1):
        c = lax.axis_index("c")
        s = lax.axis_index("s")
        wid = c * NS + s
        base = wid * pw

        def fire_events(i):
            off = base + jnp.minimum(i * CHUNK, last_off)
            off = pl.multiple_of((off >> 3) << 3, 8)
            sb = lax.rem(i, 2) * CHUNK
            pltpu.async_copy(exr.at[pl.ds(off, CHUNK)],
                             xv.at[pl.ds(sb, CHUNK)], sem_ev)
            pltpu.async_copy(eyr.at[pl.ds(off, CHUNK)],
                             yv.at[pl.ds(sb, CHUNK)], sem_ev)
            pltpu.async_copy(epr.at[pl.ds(off, CHUNK)],
                             pv.at[pl.ds(sb, CHUNK)], sem_ev)

        def wait_events():
            pltpu.make_async_copy(exr.at[pl.ds(0, CHUNK)],
                                  xv.at[pl.ds(0, CHUNK)], sem_ev).wait()
            pltpu.make_async_copy(eyr.at[pl.ds(0, CHUNK)],
                                  yv.at[pl.ds(0, CHUNK)], sem_ev).wait()
            pltpu.make_async_copy(epr.at[pl.ds(0, CHUNK)],
                                  pv.at[pl.ds(0, CHUNK)], sem_ev).wait()

        fire_events(0)

        z16 = jnp.zeros((16,), jnp.float32)

        def zfill(k, _):
            zbuf[pl.ds(k * 16, 16)] = z16
            return 0

        lax.fori_loop(0, ZCH // 16, zfill, 0)

        def zfire(k, _):
            pltpu.async_copy(zbuf, acc.at[pl.ds(s * SLICE + k * ZCH, ZCH)],
                             sem_sc0)
            return 0

        lax.fori_loop(0, SLICE // ZCH, zfire, 0)

        def zdrain(k, _):
            pltpu.make_async_copy(zbuf, acc.at[pl.ds(0, ZCH)],
                                  sem_sc0).wait()
            return 0

        lax.fori_loop(0, SLICE // ZCH, zdrain, 0)
        plsc.subcore_barrier()

        def drain_scatters(idxb, valb, sem):
            def drain(j, _):
                pltpu.make_async_copy(valb.at[0], acc.at[idxb.at[0]],
                                      sem).wait()
                return 0

            lax.fori_loop(0, ROWS, drain, 0)

        def do_chunk(i, idxb, valb, sem, first):
            sb = lax.rem(i, 2) * CHUNK
            wait_events()

            @pl.when(i + 1 < nch)
            def _():
                fire_events(i + 1)

            if not first:
                @pl.when(i >= 2)
                def _():
                    drain_scatters(idxb, valb, sem)
            else:
                drain_scatters(idxb, valb, sem)

            def row(j, _):
                for t in range(8):
                    sl = pl.ds(sb + j * 128 + t * 16, 16)
                    xx = xv[sl]
                    yy = yv[sl]
                    pp = pv[sl]
                    idx = xx * HP + yy
                    val = jnp.where(pp == 0, jnp.float32(1.0),
                                    jnp.float32(BIG))
                    idxb[j, pl.ds(t * 16, 16)] = idx
                    valb[j, pl.ds(t * 16, 16)] = val
                pltpu.async_copy(valb.at[j], acc.at[idxb.at[j]], sem,
                                 add=True)
                return 0

            lax.fori_loop(0, ROWS, row, 0)

        def pair(k, _):
            do_chunk(2 * k, idxb0, valb0, sem_sc0, first=False)
            do_chunk(2 * k + 1, idxb1, valb1, sem_sc1, first=False)
            return 0

        lax.fori_loop(0, nch // 2, pair, 0)
        drain_scatters(idxb0, valb0, sem_sc0)
        drain_scatters(idxb1, valb1, sem_sc1)
        plsc.subcore_barrier()

        rpw = W // NS

        def prow(r, _):
            pltpu.async_copy(acc.at[pl.ds((s * rpw + r) * HP, HP)],
                             outr.at[c, s * rpw + r], sem_ev)
            return 0

        lax.fori_loop(0, rpw, prow, 0)

        def drow(r, _):
            pltpu.make_async_copy(acc.at[pl.ds(0, HP)], outr.at[c, 0],
                                  sem_ev).wait()
            return 0

        lax.fori_loop(0, rpw, drow, 0)

    return body(events_x, events_y, events_polarity)


def _merge_body(a_ref, o_ref):
    t = a_ref[0, :, :H] + a_ref[1, :, :H]
    o_ref[...] = jnp.where(
        t >= jnp.float32(BIG), 255, jnp.where(t > 0, 0, 127)
    ).astype(jnp.uint8)


def _merge(acc):
    bx = 1280
    return pl.pallas_call(
        _merge_body,
        grid=(W // bx,),
        in_specs=[pl.BlockSpec((NC, bx, HP), lambda i: (0, i, 0))],
        out_specs=pl.BlockSpec((bx, H), lambda i: (i, 0)),
        out_shape=jax.ShapeDtypeStruct((W, H), jnp.uint8),
    )(acc)


def kernel(events_x, events_y, events_polarity):
    acc = _sc_scatter(events_x, events_y, events_polarity)
    return _merge(acc)


# speedup vs baseline: 1.0092x; 1.0092x over previous
import functools

import jax
import jax.numpy as jnp
from jax import lax
from jax.experimental import pallas as pl
from jax.experimental.pallas import tpu as pltpu
from jax.experimental.pallas import tpu_sc as plsc

W = 1280
H = 720
HP = 768
WHP = W * HP
BIG = float(1 << 22)

NC = 2
NS = 16
NW = NC * NS

CHUNK = 4096
ROWS = CHUNK // 128
SLICE = WHP // NS
ZCH = 7680


def _sc_scatter(events_x, events_y, events_polarity):
    n = events_x.shape[0]
    pw = n // NW
    nch = -(-pw // CHUNK)
    last_off = pw - CHUNK

    mesh = plsc.VectorSubcoreMesh(core_axis_name="c", subcore_axis_name="s")

    @functools.partial(
        pl.kernel,
        mesh=mesh,
        out_type=jax.ShapeDtypeStruct((NC, W, HP), jnp.float32),
        scratch_types=[
            pltpu.VMEM((2 * CHUNK,), jnp.int32),
            pltpu.VMEM((2 * CHUNK,), jnp.int32),
            pltpu.VMEM((2 * CHUNK,), jnp.int32),
            pltpu.VMEM((ROWS, 128), jnp.int32),
            pltpu.VMEM((ROWS, 128), jnp.float32),
            pltpu.VMEM((ROWS, 128), jnp.int32),
            pltpu.VMEM((ROWS, 128), jnp.float32),
            pltpu.VMEM((ZCH,), jnp.float32),
            pltpu.VMEM_SHARED((WHP,), jnp.float32),
            pltpu.SemaphoreType.DMA,
            pltpu.SemaphoreType.DMA,
            pltpu.SemaphoreType.DMA,
        ],
    )
    def body(exr, eyr, epr, outr, xv, yv, pv, idxb0, valb0, idxb1, valb1,
             zbuf, acc, sem_ev, sem_sc0, sem_sc1):
        c = lax.axis_index("c")
        s = lax.axis_index("s")
        wid = c * NS + s
        base = wid * pw

        def fire_events(i):
            off = base + jnp.minimum(i * CHUNK, last_off)
            off = pl.multiple_of((off >> 3) << 3, 8)
            sb = lax.rem(i, 2) * CHUNK
            pltpu.async_copy(exr.at[pl.ds(off, CHUNK)],
                             xv.at[pl.ds(sb, CHUNK)], sem_ev)
            pltpu.async_copy(eyr.at[pl.ds(off, CHUNK)],
                             yv.at[pl.ds(sb, CHUNK)], sem_ev)
            pltpu.async_copy(epr.at[pl.ds(off, CHUNK)],
                             pv.at[pl.ds(sb, CHUNK)], sem_ev)

        def wait_events():
            pltpu.make_async_copy(exr.at[pl.ds(0, CHUNK)],
                                  xv.at[pl.ds(0, CHUNK)], sem_ev).wait()
            pltpu.make_async_copy(eyr.at[pl.ds(0, CHUNK)],
                                  yv.at[pl.ds(0, CHUNK)], sem_ev).wait()
            pltpu.make_async_copy(epr.at[pl.ds(0, CHUNK)],
                                  pv.at[pl.ds(0, CHUNK)], sem_ev).wait()

        fire_events(0)

        z16 = jnp.zeros((16,), jnp.float32)

        def zfill(k, _):
            zbuf[pl.ds(k * 16, 16)] = z16
            return 0

        lax.fori_loop(0, ZCH // 16, zfill, 0)

        def zfire(k, _):
            pltpu.async_copy(zbuf, acc.at[pl.ds(s * SLICE + k * ZCH, ZCH)],
                             sem_sc0)
            return 0

        lax.fori_loop(0, SLICE // ZCH, zfire, 0)

        def zdrain(k, _):
            pltpu.make_async_copy(zbuf, acc.at[pl.ds(0, ZCH)],
                                  sem_sc0).wait()
            return 0

        lax.fori_loop(0, SLICE // ZCH, zdrain, 0)
        plsc.subcore_barrier()

        def drain_scatters(idxb, valb, sem):
            def drain(j, _):
                pltpu.make_async_copy(valb.at[0], acc.at[idxb.at[0]],
                                      sem).wait()
                return 0

            lax.fori_loop(0, ROWS, drain, 0)

        def do_chunk(i, idxb, valb, sem, first):
            sb = lax.rem(i, 2) * CHUNK
            wait_events()

            @pl.when(i + 1 < nch)
            def _():
                fire_events(i + 1)

            if not first:
                @pl.when(i >= 2)
                def _():
                    drain_scatters(idxb, valb, sem)
            else:
                drain_scatters(idxb, valb, sem)

            def row(j, _):
                for t in range(8):
                    sl = pl.ds(sb + j * 128 + t * 16, 16)
                    xx = xv[sl]
                    yy = yv[sl]
                    pp = pv[sl]
                    idx = xx * HP + yy
                    val = jnp.where(pp == 0, jnp.float32(1.0),
                                    jnp.float32(BIG))
                    idxb[j, pl.ds(t * 16, 16)] = idx
                    valb[j, pl.ds(t * 16, 16)] = val
                pltpu.async_copy(valb.at[j], acc.at[idxb.at[j]], sem,
                                 add=True)
                return 0

            lax.fori_loop(0, ROWS, row, 0)

        def pair(k, _):
            do_chunk(2 * k, idxb0, valb0, sem_sc0, first=False)
            do_chunk(2 * k + 1, idxb1, valb1, sem_sc1, first=False)
            return 0

        lax.fori_loop(0, nch // 2, pair, 0)
        drain_scatters(idxb0, valb0, sem_sc0)
        drain_scatters(idxb1, valb1, sem_sc1)
        plsc.subcore_barrier()

        rpw = W // NS

        def prow(r, _):
            pltpu.async_copy(acc.at[pl.ds((s * rpw + r) * HP, HP)],
                             outr.at[c, s * rpw + r], sem_ev)
            return 0

        lax.fori_loop(0, rpw, prow, 0)

        def drow(r, _):
            pltpu.make_async_copy(acc.at[pl.ds(0, HP)], outr.at[c, 0],
                                  sem_ev).wait()
            return 0

        lax.fori_loop(0, rpw, drow, 0)

    return body(events_x, events_y, events_polarity)


def _merge_body(a_ref, o_ref):
    t = a_ref[0, :, :H] + a_ref[1, :, :H]
    o_ref[...] = jnp.where(
        t >= jnp.float32(BIG), 255, jnp.where(t > 0, 0, 127)
    ).astype(jnp.uint8)


def _merge(acc):
    bx = 640
    return pl.pallas_call(
        _merge_body,
        grid=(W // bx,),
        in_specs=[pl.BlockSpec((NC, bx, HP), lambda i: (0, i, 0))],
        out_specs=pl.BlockSpec((bx, H), lambda i: (i, 0)),
        out_shape=jax.ShapeDtypeStruct((W, H), jnp.uint8),
    )(acc)


def kernel(events_x, events_y, events_polarity):
    acc = _sc_scatter(events_x, events_y, events_polarity)
    return _merge(acc)
